# baseline (speedup 1.0000x reference)
import functools

import jax
import jax.numpy as jnp
from jax import lax
from jax.experimental import pallas as pl
from jax.experimental.pallas import tpu as pltpu
from jax.experimental.pallas import tpu_sc as plsc

_NC = 2
_NS = 16
_NW = _NC * _NS
_CH = 128

_INV_SQRT2 = 0.7071067811865476


def _gelu(v):
    return 0.5 * v * (1.0 + lax.erf(v * _INV_SQRT2))


def _gather_body(nchunks, base_t, extra, row_off, x_hbm, src2d_hbm, xj_hbm,
                 idx_a, idx_b, rows_a, rows_b, sia, sib, sga, sgb, soa, sob):
    c = lax.axis_index("c")
    s = lax.axis_index("s")
    w = s * _NC + c

    def irow(t):
        return jnp.minimum(w + _NW * t, nchunks - 1) + row_off

    def load_idx(buf, sem, t):
        pltpu.async_copy(src2d_hbm.at[irow(t)], buf, sem)

    def wait_idx(buf, sem):
        pltpu.make_async_copy(src2d_hbm.at[0], buf, sem).wait()

    def out_copy(buf, sem, t):
        return pltpu.async_copy(
            buf, xj_hbm.at[pl.ds((w + _NW * t) * _CH, _CH)], sem)

    load_idx(idx_a, sia, 0)
    load_idx(idx_b, sib, 1)

    def pair(q, carry):
        t0 = 2 * q
        wait_idx(idx_a, sia)
        wait_idx(idx_b, sib)
        ga = pltpu.async_copy(x_hbm.at[idx_a], rows_a, sga)
        gb = pltpu.async_copy(x_hbm.at[idx_b], rows_b, sgb)
        ga.wait()
        load_idx(idx_a, sia, t0 + 2)
        oa = out_copy(rows_a, soa, t0)
        gb.wait()
        load_idx(idx_b, sib, t0 + 3)
        ob = out_copy(rows_b, sob, t0 + 1)
        oa.wait()
        ob.wait()
        return carry

    lax.fori_loop(0, base_t // 2, pair, 0)

    wait_idx(idx_a, sia)
    ga = pltpu.async_copy(x_hbm.at[idx_a], rows_a, sga)
    ga.wait()
    oa = out_copy(rows_a, soa, base_t - 1)
    wait_idx(idx_b, sib)

    @pl.when(w < extra)
    def _():
        gb = pltpu.async_copy(x_hbm.at[idx_b], rows_b, sgb)
        gb.wait()
        ob = out_copy(rows_b, sob, base_t)
        ob.wait()

    oa.wait()


def _gather(x, ei2d):
    e = (ei2d.shape[0] // 2) * ei2d.shape[1]
    h = x.shape[1]
    nchunks = e // _CH
    base_t, extra = divmod(nchunks, _NW)
    mesh = plsc.VectorSubcoreMesh(core_axis_name="c", subcore_axis_name="s")
    scratch = [
        pltpu.VMEM((_CH,), jnp.int32),
        pltpu.VMEM((_CH,), jnp.int32),
        pltpu.VMEM((_CH, h), jnp.float32),
        pltpu.VMEM((_CH, h), jnp.float32),
    ] + [pltpu.SemaphoreType.DMA] * 6
    k = functools.partial(
        pl.kernel,
        mesh=mesh,
        out_type=jax.ShapeDtypeStruct((e, h), jnp.float32),
        scratch_types=scratch,
        compiler_params=pltpu.CompilerParams(use_tc_tiling_on_sc=False),
    )(functools.partial(_gather_body, nchunks, base_t, extra, 0))
    return k(x, ei2d)


def _scatter_body(nchunks, base_t, extra, nz, row_off, msg_hbm, dst2d_hbm,
                  zeros_hbm, out_hbm, idx_a, idx_b, msg_a, msg_b, accum,
                  sia, sib, sma, smb):
    c = lax.axis_index("c")
    s = lax.axis_index("s")
    w = s * _NC + c

    pltpu.sync_copy(zeros_hbm.at[pl.ds(s * nz, nz)], accum.at[pl.ds(s * nz, nz)])
    plsc.subcore_barrier()

    def irow(t):
        return jnp.minimum(w + _NW * t, nchunks - 1)

    def load(t, ibuf, isem, mbuf, msem):
        r = irow(t)
        pltpu.async_copy(dst2d_hbm.at[r + row_off], ibuf, isem)
        pltpu.async_copy(msg_hbm.at[pl.ds(r * _CH, _CH)], mbuf, msem)

    def wait_loads(ibuf, isem, mbuf, msem):
        pltpu.make_async_copy(dst2d_hbm.at[0], ibuf, isem).wait()
        pltpu.make_async_copy(msg_hbm.at[pl.ds(0, _CH)], mbuf, msem).wait()

    load(0, idx_a, sia, msg_a, sma)
    load(1, idx_b, sib, msg_b, smb)

    def pair(q, carry):
        t0 = 2 * q
        wait_loads(idx_a, sia, msg_a, sma)
        pltpu.sync_copy(msg_a, accum.at[idx_a], add=True)
        load(t0 + 2, idx_a, sia, msg_a, sma)
        wait_loads(idx_b, sib, msg_b, smb)
        pltpu.sync_copy(msg_b, accum.at[idx_b], add=True)
        load(t0 + 3, idx_b, sib, msg_b, smb)
        return carry

    lax.fori_loop(0, base_t // 2, pair, 0)

    wait_loads(idx_a, sia, msg_a, sma)
    pltpu.sync_copy(msg_a, accum.at[idx_a], add=True)
    wait_loads(idx_b, sib, msg_b, smb)

    @pl.when(w < extra)
    def _():
        pltpu.sync_copy(msg_b, accum.at[idx_b], add=True)

    plsc.subcore_barrier()
    n = nz * _NS
    pltpu.sync_copy(accum.at[pl.ds(s * nz, nz)],
                    out_hbm.at[pl.ds(c * n + s * nz, nz)])


def _scatter_add(msg, ei2d, n_nodes):
    e, h = msg.shape
    nchunks = e // _CH
    base_t, extra = divmod(nchunks, _NW)
    nz = n_nodes // _NS
    zeros = jnp.zeros((n_nodes, h), jnp.float32)
    mesh = plsc.VectorSubcoreMesh(core_axis_name="c", subcore_axis_name="s")
    scratch = [
        pltpu.VMEM((_CH,), jnp.int32),
        pltpu.VMEM((_CH,), jnp.int32),
        pltpu.VMEM((_CH, h), jnp.float32),
        pltpu.VMEM((_CH, h), jnp.float32),
        pltpu.VMEM_SHARED((n_nodes, h), jnp.float32),
    ] + [pltpu.SemaphoreType.DMA] * 4
    k = functools.partial(
        pl.kernel,
        mesh=mesh,
        out_type=jax.ShapeDtypeStruct((_NC * n_nodes, h), jnp.float32),
        scratch_types=scratch,
        compiler_params=pltpu.CompilerParams(use_tc_tiling_on_sc=False),
    )(functools.partial(_scatter_body, nchunks, base_t, extra, nz, nchunks))
    return k(msg, ei2d, zeros)


def _msg_body(h2, ea_ref, xj_ref, w1_ref, b1_ref, w2_ref, b2m_ref, r_ref,
              msg_ref):
    h = _gelu(jnp.dot(ea_ref[...], w1_ref[...],
                      preferred_element_type=jnp.float32) + b1_ref[...])
    hb = h.astype(jnp.bfloat16)
    xjb = xj_ref[...].astype(jnp.bfloat16)
    we = jnp.dot(hb, w2_ref[...], preferred_element_type=jnp.float32)
    xr = jnp.dot(xjb, r_ref[...], preferred_element_type=jnp.float32)
    prod = we * xr
    n = h2 * h2
    while n > h2:
        n //= 2
        prod = prod[:, :n] + prod[:, n:2 * n]
    msg_ref[...] = prod + jnp.dot(xjb, b2m_ref[...],
                                  preferred_element_type=jnp.float32)


def _messages(edge_attr, xj, w1, b1, w2, b2):
    e, d = edge_attr.shape
    h = xj.shape[1]
    blk = 4000
    grid = (e // blk,)
    rep = jnp.repeat(jnp.eye(h, dtype=jnp.bfloat16), h, axis=1)
    w2b = w2.astype(jnp.bfloat16)
    return pl.pallas_call(
        functools.partial(_msg_body, h),
        grid=grid,
        in_specs=[
            pl.BlockSpec((blk, d), lambda i: (i, 0)),
            pl.BlockSpec((blk, h), lambda i: (i, 0)),
            pl.BlockSpec((d, h), lambda i: (0, 0)),
            pl.BlockSpec((1, h), lambda i: (0, 0)),
            pl.BlockSpec((h, h * h), lambda i: (0, 0)),
            pl.BlockSpec((h, h), lambda i: (0, 0)),
            pl.BlockSpec((h, h * h), lambda i: (0, 0)),
        ],
        out_specs=pl.BlockSpec((blk, h), lambda i: (i, 0)),
        out_shape=jax.ShapeDtypeStruct((e, h), jnp.float32),
    )(edge_attr, xj, w1, b1.reshape(1, h), w2b,
      b2.reshape(h, h).astype(jnp.bfloat16), rep)


def _update_body(x_ref, p0_ref, p1_ref, root_ref, bias_ref, g_ref, b_ref,
                 out_ref):
    xb = x_ref[...]
    a = (p0_ref[...] + p1_ref[...]
         + jnp.dot(xb, root_ref[...], preferred_element_type=jnp.float32)
         + bias_ref[...])
    hh = _gelu(a) + xb
    mu = jnp.mean(hh, axis=1, keepdims=True)
    dlt = hh - mu
    var = jnp.mean(dlt * dlt, axis=1, keepdims=True)
    out_ref[...] = g_ref[...] * dlt * lax.rsqrt(var + 1e-5) + b_ref[...]


def _update(x, parts, root, bias, gamma, beta):
    n, h = x.shape
    blk = 2000
    grid = (n // blk,)
    off = n // blk
    row = lambda i: (i, 0)
    fix = lambda i: (0, 0)
    return pl.pallas_call(
        _update_body,
        grid=grid,
        in_specs=[
            pl.BlockSpec((blk, h), row),
            pl.BlockSpec((blk, h), row),
            pl.BlockSpec((blk, h), lambda i: (i + off, 0)),
            pl.BlockSpec((h, h), fix),
            pl.BlockSpec((1, h), fix),
            pl.BlockSpec((1, h), fix),
            pl.BlockSpec((1, h), fix),
        ],
        out_specs=pl.BlockSpec((blk, h), row),
        out_shape=jax.ShapeDtypeStruct((n, h), jnp.float32),
    )(x, parts, parts, root, bias.reshape(1, h), gamma.reshape(1, h),
      beta.reshape(1, h))


def kernel(x, edge_index, edge_attr, W1, b1, W2, b2, root, bias, gamma, beta):
    n = x.shape[0]
    e = edge_index.shape[1]
    ei2d = edge_index.reshape(2 * (e // _CH), _CH)
    xj = _gather(x, ei2d)
    msg = _messages(edge_attr, xj, W1, b1, W2, b2)
    parts = _scatter_add(msg, ei2d, n)
    return _update(x, parts, root, bias, gamma, beta)


# speedup vs baseline: 1.0406x; 1.0406x over previous
import functools

import jax
import jax.numpy as jnp
from jax import lax
from jax.experimental import pallas as pl
from jax.experimental.pallas import tpu as pltpu
from jax.experimental.pallas import tpu_sc as plsc

_NC = 2
_NS = 16
_NW = _NC * _NS
_CH = 128

_INV_SQRT2 = 0.7071067811865476


def _gelu(v):
    return 0.5 * v * (1.0 + lax.erf(v * _INV_SQRT2))


def _gather_body(nchunks, base_t, extra, row_off, x_hbm, src2d_hbm, xj_hbm,
                 idx_a, idx_b, rows_a, rows_b, sia, sib, sga, sgb, soa, sob):
    c = lax.axis_index("c")
    s = lax.axis_index("s")
    w = s * _NC + c

    def irow(t):
        return jnp.minimum(w + _NW * t, nchunks - 1) + row_off

    def load_idx(buf, sem, t):
        pltpu.async_copy(src2d_hbm.at[irow(t)], buf, sem)

    def wait_idx(buf, sem):
        pltpu.make_async_copy(src2d_hbm.at[0], buf, sem).wait()

    def out_copy(buf, sem, t):
        return pltpu.async_copy(
            buf, xj_hbm.at[pl.ds((w + _NW * t) * _CH, _CH)], sem)

    load_idx(idx_a, sia, 0)
    load_idx(idx_b, sib, 1)

    def pair(q, carry):
        t0 = 2 * q
        wait_idx(idx_a, sia)
        wait_idx(idx_b, sib)
        ga = pltpu.async_copy(x_hbm.at[idx_a], rows_a, sga)
        gb = pltpu.async_copy(x_hbm.at[idx_b], rows_b, sgb)
        ga.wait()
        load_idx(idx_a, sia, t0 + 2)
        oa = out_copy(rows_a, soa, t0)
        gb.wait()
        load_idx(idx_b, sib, t0 + 3)
        ob = out_copy(rows_b, sob, t0 + 1)
        oa.wait()
        ob.wait()
        return carry

    lax.fori_loop(0, base_t // 2, pair, 0)

    wait_idx(idx_a, sia)
    ga = pltpu.async_copy(x_hbm.at[idx_a], rows_a, sga)
    ga.wait()
    oa = out_copy(rows_a, soa, base_t - 1)
    wait_idx(idx_b, sib)

    @pl.when(w < extra)
    def _():
        gb = pltpu.async_copy(x_hbm.at[idx_b], rows_b, sgb)
        gb.wait()
        ob = out_copy(rows_b, sob, base_t)
        ob.wait()

    oa.wait()


def _gather(x, ei2d):
    e = (ei2d.shape[0] // 2) * ei2d.shape[1]
    h = x.shape[1]
    nchunks = e // _CH
    base_t, extra = divmod(nchunks, _NW)
    mesh = plsc.VectorSubcoreMesh(core_axis_name="c", subcore_axis_name="s")
    scratch = [
        pltpu.VMEM((_CH,), jnp.int32),
        pltpu.VMEM((_CH,), jnp.int32),
        pltpu.VMEM((_CH, h), jnp.float32),
        pltpu.VMEM((_CH, h), jnp.float32),
    ] + [pltpu.SemaphoreType.DMA] * 6
    k = functools.partial(
        pl.kernel,
        mesh=mesh,
        out_type=jax.ShapeDtypeStruct((e, h), jnp.float32),
        scratch_types=scratch,
        compiler_params=pltpu.CompilerParams(use_tc_tiling_on_sc=False),
    )(functools.partial(_gather_body, nchunks, base_t, extra, 0))
    return k(x, ei2d)


def _scatter_body(nchunks, base_t, extra, nz, row_off, msg_hbm, dst2d_hbm,
                  zeros_hbm, out_hbm, idx_a, idx_b, msg_a, msg_b, accum,
                  sia, sib, sma, smb):
    c = lax.axis_index("c")
    s = lax.axis_index("s")
    w = s * _NC + c

    pltpu.sync_copy(zeros_hbm.at[pl.ds(s * nz, nz)], accum.at[pl.ds(s * nz, nz)])
    plsc.subcore_barrier()

    def irow(t):
        return jnp.minimum(w + _NW * t, nchunks - 1)

    def load(t, ibuf, isem, mbuf, msem):
        r = irow(t)
        pltpu.async_copy(dst2d_hbm.at[r + row_off], ibuf, isem)
        pltpu.async_copy(msg_hbm.at[pl.ds(r * _CH, _CH)], mbuf, msem)

    def wait_loads(ibuf, isem, mbuf, msem):
        pltpu.make_async_copy(dst2d_hbm.at[0], ibuf, isem).wait()
        pltpu.make_async_copy(msg_hbm.at[pl.ds(0, _CH)], mbuf, msem).wait()

    load(0, idx_a, sia, msg_a, sma)
    load(1, idx_b, sib, msg_b, smb)

    def pair(q, carry):
        t0 = 2 * q
        wait_loads(idx_a, sia, msg_a, sma)
        pltpu.sync_copy(msg_a, accum.at[idx_a], add=True)
        load(t0 + 2, idx_a, sia, msg_a, sma)
        wait_loads(idx_b, sib, msg_b, smb)
        pltpu.sync_copy(msg_b, accum.at[idx_b], add=True)
        load(t0 + 3, idx_b, sib, msg_b, smb)
        return carry

    lax.fori_loop(0, base_t // 2, pair, 0)

    wait_loads(idx_a, sia, msg_a, sma)
    pltpu.sync_copy(msg_a, accum.at[idx_a], add=True)
    wait_loads(idx_b, sib, msg_b, smb)

    @pl.when(w < extra)
    def _():
        pltpu.sync_copy(msg_b, accum.at[idx_b], add=True)

    plsc.subcore_barrier()
    n = nz * _NS
    pltpu.sync_copy(accum.at[pl.ds(s * nz, nz)],
                    out_hbm.at[pl.ds(c * n + s * nz, nz)])


def _scatter_add(msg, ei2d, n_nodes):
    e, h = msg.shape
    nchunks = e // _CH
    base_t, extra = divmod(nchunks, _NW)
    nz = n_nodes // _NS
    zeros = jnp.zeros((n_nodes, h), jnp.float32)
    mesh = plsc.VectorSubcoreMesh(core_axis_name="c", subcore_axis_name="s")
    scratch = [
        pltpu.VMEM((_CH,), jnp.int32),
        pltpu.VMEM((_CH,), jnp.int32),
        pltpu.VMEM((_CH, h), jnp.float32),
        pltpu.VMEM((_CH, h), jnp.float32),
        pltpu.VMEM_SHARED((n_nodes, h), jnp.float32),
    ] + [pltpu.SemaphoreType.DMA] * 4
    k = functools.partial(
        pl.kernel,
        mesh=mesh,
        out_type=jax.ShapeDtypeStruct((_NC * n_nodes, h), jnp.float32),
        scratch_types=scratch,
        compiler_params=pltpu.CompilerParams(use_tc_tiling_on_sc=False),
    )(functools.partial(_scatter_body, nchunks, base_t, extra, nz, nchunks))
    return k(msg, ei2d, zeros)


def _msg_body(h2, ea_ref, xj_ref, w1_ref, b1_ref, w2_ref, b2m_ref, r_ref,
              msg_ref):
    h = _gelu(jnp.dot(ea_ref[...].astype(jnp.bfloat16), w1_ref[...],
                      preferred_element_type=jnp.float32) + b1_ref[...])
    hb = h.astype(jnp.bfloat16)
    xjb = xj_ref[...].astype(jnp.bfloat16)
    ck = 256
    acc = None
    for c in range(0, h2 * h2, ck):
        we = jnp.dot(hb, w2_ref[:, c:c + ck],
                     preferred_element_type=jnp.float32)
        xr = jnp.dot(xjb, r_ref[:, c:c + ck],
                     preferred_element_type=jnp.float32)
        p = we * xr
        p = p[:, :128] + p[:, 128:]
        acc = p if acc is None else acc + p
    acc = acc[:, :64] + acc[:, 64:]
    msg_ref[...] = (acc[:, :32] + acc[:, 32:]
                    + jnp.dot(xjb, b2m_ref[...],
                              preferred_element_type=jnp.float32))


def _messages(edge_attr, xj, w1, b1, w2, b2):
    e, d = edge_attr.shape
    h = xj.shape[1]
    blk = 4000
    grid = (e // blk,)
    rep = jnp.repeat(jnp.eye(h, dtype=jnp.bfloat16), h, axis=1)
    w2b = w2.astype(jnp.bfloat16)
    return pl.pallas_call(
        functools.partial(_msg_body, h),
        grid=grid,
        in_specs=[
            pl.BlockSpec((blk, d), lambda i: (i, 0)),
            pl.BlockSpec((blk, h), lambda i: (i, 0)),
            pl.BlockSpec((d, h), lambda i: (0, 0)),
            pl.BlockSpec((1, h), lambda i: (0, 0)),
            pl.BlockSpec((h, h * h), lambda i: (0, 0)),
            pl.BlockSpec((h, h), lambda i: (0, 0)),
            pl.BlockSpec((h, h * h), lambda i: (0, 0)),
        ],
        out_specs=pl.BlockSpec((blk, h), lambda i: (i, 0)),
        out_shape=jax.ShapeDtypeStruct((e, h), jnp.float32),
    )(edge_attr, xj, w1.astype(jnp.bfloat16), b1.reshape(1, h), w2b,
      b2.reshape(h, h).astype(jnp.bfloat16), rep)


def _update_body(x_ref, p0_ref, p1_ref, root_ref, bias_ref, g_ref, b_ref,
                 out_ref):
    xb = x_ref[...]
    a = (p0_ref[...] + p1_ref[...]
         + jnp.dot(xb, root_ref[...], preferred_element_type=jnp.float32)
         + bias_ref[...])
    hh = _gelu(a) + xb
    mu = jnp.mean(hh, axis=1, keepdims=True)
    dlt = hh - mu
    var = jnp.mean(dlt * dlt, axis=1, keepdims=True)
    out_ref[...] = g_ref[...] * dlt * lax.rsqrt(var + 1e-5) + b_ref[...]


def _update(x, parts, root, bias, gamma, beta):
    n, h = x.shape
    blk = 2000
    grid = (n // blk,)
    off = n // blk
    row = lambda i: (i, 0)
    fix = lambda i: (0, 0)
    return pl.pallas_call(
        _update_body,
        grid=grid,
        in_specs=[
            pl.BlockSpec((blk, h), row),
            pl.BlockSpec((blk, h), row),
            pl.BlockSpec((blk, h), lambda i: (i + off, 0)),
            pl.BlockSpec((h, h), fix),
            pl.BlockSpec((1, h), fix),
            pl.BlockSpec((1, h), fix),
            pl.BlockSpec((1, h), fix),
        ],
        out_specs=pl.BlockSpec((blk, h), row),
        out_shape=jax.ShapeDtypeStruct((n, h), jnp.float32),
    )(x, parts, parts, root, bias.reshape(1, h), gamma.reshape(1, h),
      beta.reshape(1, h))


def kernel(x, edge_index, edge_attr, W1, b1, W2, b2, root, bias, gamma, beta):
    n = x.shape[0]
    e = edge_index.shape[1]
    ei2d = edge_index.reshape(2 * (e // _CH), _CH)
    xj = _gather(x, ei2d)
    msg = _messages(edge_attr, xj, W1, b1, W2, b2)
    parts = _scatter_add(msg, ei2d, n)
    return _update(x, parts, root, bias, gamma, beta)


# speedup vs baseline: 1.0444x; 1.0037x over previous
import functools

import jax
import jax.numpy as jnp
from jax import lax
from jax.experimental import pallas as pl
from jax.experimental.pallas import tpu as pltpu
from jax.experimental.pallas import tpu_sc as plsc

_NC = 2
_NS = 16
_NW = _NC * _NS
_CH = 128

_INV_SQRT2 = 0.7071067811865476


def _gelu(v):
    return 0.5 * v * (1.0 + lax.erf(v * _INV_SQRT2))


def _gather_body(nchunks, base_t, extra, row_off, x_hbm, src2d_hbm, xj_hbm,
                 idx_a, idx_b, rows_a, rows_b, sia, sib, sga, sgb, soa, sob):
    c = lax.axis_index("c")
    s = lax.axis_index("s")
    w = s * _NC + c

    def irow(t):
        return jnp.minimum(w + _NW * t, nchunks - 1) + row_off

    def load_idx(buf, sem, t):
        pltpu.async_copy(src2d_hbm.at[irow(t)], buf, sem)

    def wait_idx(buf, sem):
        pltpu.make_async_copy(src2d_hbm.at[0], buf, sem).wait()

    def out_copy(buf, sem, t):
        return pltpu.async_copy(
            buf, xj_hbm.at[pl.ds((w + _NW * t) * _CH, _CH)], sem)

    load_idx(idx_a, sia, 0)
    load_idx(idx_b, sib, 1)

    def pair(q, carry):
        t0 = 2 * q
        wait_idx(idx_a, sia)
        wait_idx(idx_b, sib)
        ga = pltpu.async_copy(x_hbm.at[idx_a], rows_a, sga)
        gb = pltpu.async_copy(x_hbm.at[idx_b], rows_b, sgb)
        ga.wait()
        load_idx(idx_a, sia, t0 + 2)
        oa = out_copy(rows_a, soa, t0)
        gb.wait()
        load_idx(idx_b, sib, t0 + 3)
        ob = out_copy(rows_b, sob, t0 + 1)
        oa.wait()
        ob.wait()
        return carry

    lax.fori_loop(0, base_t // 2, pair, 0)

    wait_idx(idx_a, sia)
    ga = pltpu.async_copy(x_hbm.at[idx_a], rows_a, sga)
    ga.wait()
    oa = out_copy(rows_a, soa, base_t - 1)
    wait_idx(idx_b, sib)

    @pl.when(w < extra)
    def _():
        gb = pltpu.async_copy(x_hbm.at[idx_b], rows_b, sgb)
        gb.wait()
        ob = out_copy(rows_b, sob, base_t)
        ob.wait()

    oa.wait()


def _gather(x, ei2d):
    e = (ei2d.shape[0] // 2) * ei2d.shape[1]
    h = x.shape[1]
    nchunks = e // _CH
    base_t, extra = divmod(nchunks, _NW)
    mesh = plsc.VectorSubcoreMesh(core_axis_name="c", subcore_axis_name="s")
    scratch = [
        pltpu.VMEM((_CH,), jnp.int32),
        pltpu.VMEM((_CH,), jnp.int32),
        pltpu.VMEM((_CH, h), jnp.float32),
        pltpu.VMEM((_CH, h), jnp.float32),
    ] + [pltpu.SemaphoreType.DMA] * 6
    k = functools.partial(
        pl.kernel,
        mesh=mesh,
        out_type=jax.ShapeDtypeStruct((e, h), jnp.float32),
        scratch_types=scratch,
        compiler_params=pltpu.CompilerParams(use_tc_tiling_on_sc=False),
    )(functools.partial(_gather_body, nchunks, base_t, extra, 0))
    return k(x, ei2d)


def _scatter_body(nchunks, base_t, extra, nz, row_off, msg_hbm, dst2d_hbm,
                  zeros_hbm, out_hbm, idx_a, idx_b, msg_a, msg_b, accum,
                  sia, sib, sma, smb):
    c = lax.axis_index("c")
    s = lax.axis_index("s")
    w = s * _NC + c

    pltpu.sync_copy(zeros_hbm.at[pl.ds(s * nz, nz)], accum.at[pl.ds(s * nz, nz)])
    plsc.subcore_barrier()

    def irow(t):
        return jnp.minimum(w + _NW * t, nchunks - 1)

    def load(t, ibuf, isem, mbuf, msem):
        r = irow(t)
        pltpu.async_copy(dst2d_hbm.at[r + row_off], ibuf, isem)
        pltpu.async_copy(msg_hbm.at[pl.ds(r * _CH, _CH)], mbuf, msem)

    def wait_loads(ibuf, isem, mbuf, msem):
        pltpu.make_async_copy(dst2d_hbm.at[0], ibuf, isem).wait()
        pltpu.make_async_copy(msg_hbm.at[pl.ds(0, _CH)], mbuf, msem).wait()

    load(0, idx_a, sia, msg_a, sma)
    load(1, idx_b, sib, msg_b, smb)

    def pair(q, carry):
        t0 = 2 * q
        wait_loads(idx_a, sia, msg_a, sma)
        pltpu.sync_copy(msg_a, accum.at[idx_a], add=True)
        load(t0 + 2, idx_a, sia, msg_a, sma)
        wait_loads(idx_b, sib, msg_b, smb)
        pltpu.sync_copy(msg_b, accum.at[idx_b], add=True)
        load(t0 + 3, idx_b, sib, msg_b, smb)
        return carry

    lax.fori_loop(0, base_t // 2, pair, 0)

    wait_loads(idx_a, sia, msg_a, sma)
    pltpu.sync_copy(msg_a, accum.at[idx_a], add=True)
    wait_loads(idx_b, sib, msg_b, smb)

    @pl.when(w < extra)
    def _():
        pltpu.sync_copy(msg_b, accum.at[idx_b], add=True)

    plsc.subcore_barrier()
    n = nz * _NS
    pltpu.sync_copy(accum.at[pl.ds(s * nz, nz)],
                    out_hbm.at[pl.ds(c * n + s * nz, nz)])


def _scatter_add(msg, ei2d, n_nodes):
    e, h = msg.shape
    nchunks = e // _CH
    base_t, extra = divmod(nchunks, _NW)
    nz = n_nodes // _NS
    zeros = jnp.zeros((n_nodes, h), jnp.float32)
    mesh = plsc.VectorSubcoreMesh(core_axis_name="c", subcore_axis_name="s")
    scratch = [
        pltpu.VMEM((_CH,), jnp.int32),
        pltpu.VMEM((_CH,), jnp.int32),
        pltpu.VMEM((_CH, h), jnp.float32),
        pltpu.VMEM((_CH, h), jnp.float32),
        pltpu.VMEM_SHARED((n_nodes, h), jnp.float32),
    ] + [pltpu.SemaphoreType.DMA] * 4
    k = functools.partial(
        pl.kernel,
        mesh=mesh,
        out_type=jax.ShapeDtypeStruct((_NC * n_nodes, h), jnp.float32),
        scratch_types=scratch,
        compiler_params=pltpu.CompilerParams(use_tc_tiling_on_sc=False),
    )(functools.partial(_scatter_body, nchunks, base_t, extra, nz, nchunks))
    return k(msg, ei2d, zeros)


def _msg_body(h2, ea_ref, xj_ref, w1_ref, b1_ref, w2_ref, r_ref, msg_ref):
    h = _gelu(jnp.dot(ea_ref[...].astype(jnp.bfloat16), w1_ref[...],
                      preferred_element_type=jnp.float32) + b1_ref[...])
    hb = jnp.concatenate(
        [h, jnp.ones_like(h[:, :1])], axis=1).astype(jnp.bfloat16)
    xjb = xj_ref[...].astype(jnp.bfloat16)
    ck = 256
    acc = None
    for c in range(0, h2 * h2, ck):
        we = jnp.dot(hb, w2_ref[:, c:c + ck],
                     preferred_element_type=jnp.float32)
        xr = jnp.dot(xjb, r_ref[:, c:c + ck],
                     preferred_element_type=jnp.float32)
        p = we * xr
        p = p[:, :128] + p[:, 128:]
        acc = p if acc is None else acc + p
    acc = acc[:, :64] + acc[:, 64:]
    msg_ref[...] = acc[:, :32] + acc[:, 32:]


def _messages(edge_attr, xj, w1, b1, w2, b2):
    e, d = edge_attr.shape
    h = xj.shape[1]
    blk = 4000
    grid = (e // blk,)
    rep = jnp.repeat(jnp.eye(h, dtype=jnp.bfloat16), h, axis=1)
    w2b = jnp.concatenate([w2, b2.reshape(1, h * h)],
                          axis=0).astype(jnp.bfloat16)
    return pl.pallas_call(
        functools.partial(_msg_body, h),
        grid=grid,
        in_specs=[
            pl.BlockSpec((blk, d), lambda i: (i, 0)),
            pl.BlockSpec((blk, h), lambda i: (i, 0)),
            pl.BlockSpec((d, h), lambda i: (0, 0)),
            pl.BlockSpec((1, h), lambda i: (0, 0)),
            pl.BlockSpec((h + 1, h * h), lambda i: (0, 0)),
            pl.BlockSpec((h, h * h), lambda i: (0, 0)),
        ],
        out_specs=pl.BlockSpec((blk, h), lambda i: (i, 0)),
        out_shape=jax.ShapeDtypeStruct((e, h), jnp.float32),
    )(edge_attr, xj, w1.astype(jnp.bfloat16), b1.reshape(1, h), w2b, rep)


def _update_body(x_ref, p0_ref, p1_ref, root_ref, bias_ref, g_ref, b_ref,
                 out_ref):
    xb = x_ref[...]
    a = (p0_ref[...] + p1_ref[...]
         + jnp.dot(xb, root_ref[...], preferred_element_type=jnp.float32)
         + bias_ref[...])
    hh = _gelu(a) + xb
    mu = jnp.mean(hh, axis=1, keepdims=True)
    dlt = hh - mu
    var = jnp.mean(dlt * dlt, axis=1, keepdims=True)
    out_ref[...] = g_ref[...] * dlt * lax.rsqrt(var + 1e-5) + b_ref[...]


def _update(x, parts, root, bias, gamma, beta):
    n, h = x.shape
    blk = 2000
    grid = (n // blk,)
    off = n // blk
    row = lambda i: (i, 0)
    fix = lambda i: (0, 0)
    return pl.pallas_call(
        _update_body,
        grid=grid,
        in_specs=[
            pl.BlockSpec((blk, h), row),
            pl.BlockSpec((blk, h), row),
            pl.BlockSpec((blk, h), lambda i: (i + off, 0)),
            pl.BlockSpec((h, h), fix),
            pl.BlockSpec((1, h), fix),
            pl.BlockSpec((1, h), fix),
            pl.BlockSpec((1, h), fix),
        ],
        out_specs=pl.BlockSpec((blk, h), row),
        out_shape=jax.ShapeDtypeStruct((n, h), jnp.float32),
    )(x, parts, parts, root, bias.reshape(1, h), gamma.reshape(1, h),
      beta.reshape(1, h))


def kernel(x, edge_index, edge_attr, W1, b1, W2, b2, root, bias, gamma, beta):
    n = x.shape[0]
    e = edge_index.shape[1]
    ei2d = edge_index.reshape(2 * (e // _CH), _CH)
    xj = _gather(x, ei2d)
    msg = _messages(edge_attr, xj, W1, b1, W2, b2)
    parts = _scatter_add(msg, ei2d, n)
    return _update(x, parts, root, bias, gamma, beta)


# speedup vs baseline: 1.0450x; 1.0006x over previous
import functools

import jax
import jax.numpy as jnp
from jax import lax
from jax.experimental import pallas as pl
from jax.experimental.pallas import tpu as pltpu
from jax.experimental.pallas import tpu_sc as plsc

_NC = 2
_NS = 16
_NW = _NC * _NS
_CH = 128

_INV_SQRT2 = 0.7071067811865476


def _gelu(v):
    return 0.5 * v * (1.0 + lax.erf(v * _INV_SQRT2))


def _gather_body(nchunks, base_t, extra, row_off, x_hbm, src2d_hbm, xj_hbm,
                 idx_a, idx_b, rows_a, rows_b, sia, sib, sga, sgb, soa, sob):
    c = lax.axis_index("c")
    s = lax.axis_index("s")
    w = s * _NC + c

    def irow(t):
        return jnp.minimum(w + _NW * t, nchunks - 1) + row_off

    def load_idx(buf, sem, t):
        pltpu.async_copy(src2d_hbm.at[irow(t)], buf, sem)

    def wait_idx(buf, sem):
        pltpu.make_async_copy(src2d_hbm.at[0], buf, sem).wait()

    def out_copy(buf, sem, t):
        return pltpu.async_copy(
            buf, xj_hbm.at[pl.ds((w + _NW * t) * _CH, _CH)], sem)

    load_idx(idx_a, sia, 0)
    load_idx(idx_b, sib, 1)

    def pair(q, carry):
        t0 = 2 * q
        wait_idx(idx_a, sia)
        wait_idx(idx_b, sib)
        ga = pltpu.async_copy(x_hbm.at[idx_a], rows_a, sga)
        gb = pltpu.async_copy(x_hbm.at[idx_b], rows_b, sgb)
        ga.wait()
        load_idx(idx_a, sia, t0 + 2)
        oa = out_copy(rows_a, soa, t0)
        gb.wait()
        load_idx(idx_b, sib, t0 + 3)
        ob = out_copy(rows_b, sob, t0 + 1)
        oa.wait()
        ob.wait()
        return carry

    lax.fori_loop(0, base_t // 2, pair, 0)

    wait_idx(idx_a, sia)
    ga = pltpu.async_copy(x_hbm.at[idx_a], rows_a, sga)
    ga.wait()
    oa = out_copy(rows_a, soa, base_t - 1)
    wait_idx(idx_b, sib)

    @pl.when(w < extra)
    def _():
        gb = pltpu.async_copy(x_hbm.at[idx_b], rows_b, sgb)
        gb.wait()
        ob = out_copy(rows_b, sob, base_t)
        ob.wait()

    oa.wait()


def _gather(x, ei2d):
    e = (ei2d.shape[0] // 2) * ei2d.shape[1]
    h = x.shape[1]
    nchunks = e // _CH
    base_t, extra = divmod(nchunks, _NW)
    mesh = plsc.VectorSubcoreMesh(core_axis_name="c", subcore_axis_name="s")
    scratch = [
        pltpu.VMEM((_CH,), jnp.int32),
        pltpu.VMEM((_CH,), jnp.int32),
        pltpu.VMEM((_CH, h), jnp.float32),
        pltpu.VMEM((_CH, h), jnp.float32),
    ] + [pltpu.SemaphoreType.DMA] * 6
    k = functools.partial(
        pl.kernel,
        mesh=mesh,
        out_type=jax.ShapeDtypeStruct((e, h), jnp.float32),
        scratch_types=scratch,
        compiler_params=pltpu.CompilerParams(use_tc_tiling_on_sc=False),
    )(functools.partial(_gather_body, nchunks, base_t, extra, 0))
    return k(x, ei2d)


def _scatter_body(nchunks, base_t, extra, nz, row_off, msg_hbm, dst2d_hbm,
                  zeros_hbm, out_hbm, idx_a, idx_b, msg_a, msg_b, accum,
                  sia, sib, sma, smb):
    c = lax.axis_index("c")
    s = lax.axis_index("s")
    w = s * _NC + c

    pltpu.sync_copy(zeros_hbm.at[pl.ds(s * nz, nz)], accum.at[pl.ds(s * nz, nz)])
    plsc.subcore_barrier()

    def irow(t):
        return jnp.minimum(w + _NW * t, nchunks - 1)

    def load(t, ibuf, isem, mbuf, msem):
        r = irow(t)
        pltpu.async_copy(dst2d_hbm.at[r + row_off], ibuf, isem)
        pltpu.async_copy(msg_hbm.at[pl.ds(r * _CH, _CH)], mbuf, msem)

    def wait_loads(ibuf, isem, mbuf, msem):
        pltpu.make_async_copy(dst2d_hbm.at[0], ibuf, isem).wait()
        pltpu.make_async_copy(msg_hbm.at[pl.ds(0, _CH)], mbuf, msem).wait()

    load(0, idx_a, sia, msg_a, sma)
    load(1, idx_b, sib, msg_b, smb)

    def pair(q, carry):
        t0 = 2 * q
        wait_loads(idx_a, sia, msg_a, sma)
        pltpu.sync_copy(msg_a, accum.at[idx_a], add=True)
        load(t0 + 2, idx_a, sia, msg_a, sma)
        wait_loads(idx_b, sib, msg_b, smb)
        pltpu.sync_copy(msg_b, accum.at[idx_b], add=True)
        load(t0 + 3, idx_b, sib, msg_b, smb)
        return carry

    lax.fori_loop(0, base_t // 2, pair, 0)

    wait_loads(idx_a, sia, msg_a, sma)
    pltpu.sync_copy(msg_a, accum.at[idx_a], add=True)
    wait_loads(idx_b, sib, msg_b, smb)

    @pl.when(w < extra)
    def _():
        pltpu.sync_copy(msg_b, accum.at[idx_b], add=True)

    plsc.subcore_barrier()
    n = nz * _NS
    pltpu.sync_copy(accum.at[pl.ds(s * nz, nz)],
                    out_hbm.at[pl.ds(c * n + s * nz, nz)])


def _scatter_add(msg, ei2d, n_nodes):
    e, h = msg.shape
    nchunks = e // _CH
    base_t, extra = divmod(nchunks, _NW)
    nz = n_nodes // _NS
    zeros = jnp.zeros((n_nodes, h), jnp.float32)
    mesh = plsc.VectorSubcoreMesh(core_axis_name="c", subcore_axis_name="s")
    scratch = [
        pltpu.VMEM((_CH,), jnp.int32),
        pltpu.VMEM((_CH,), jnp.int32),
        pltpu.VMEM((_CH, h), jnp.float32),
        pltpu.VMEM((_CH, h), jnp.float32),
        pltpu.VMEM_SHARED((n_nodes, h), jnp.float32),
    ] + [pltpu.SemaphoreType.DMA] * 4
    k = functools.partial(
        pl.kernel,
        mesh=mesh,
        out_type=jax.ShapeDtypeStruct((_NC * n_nodes, h), jnp.float32),
        scratch_types=scratch,
        compiler_params=pltpu.CompilerParams(use_tc_tiling_on_sc=False),
    )(functools.partial(_scatter_body, nchunks, base_t, extra, nz, nchunks))
    return k(msg, ei2d, zeros)


def _msg_body(h2, ea_ref, xj_ref, w1_ref, b1_ref, w2_ref, r_ref, msg_ref):
    h = _gelu(jnp.dot(ea_ref[...].astype(jnp.bfloat16), w1_ref[...],
                      preferred_element_type=jnp.float32) + b1_ref[...])
    hb = jnp.concatenate(
        [h, jnp.ones_like(h[:, :1])], axis=1).astype(jnp.bfloat16)
    xjb = xj_ref[...].astype(jnp.bfloat16)
    ck = 256
    acc = None
    for c in range(0, h2 * h2, ck):
        we = jnp.dot(hb, w2_ref[:, c:c + ck],
                     preferred_element_type=jnp.float32)
        xr = jnp.dot(xjb, r_ref[:, c:c + ck],
                     preferred_element_type=jnp.float32)
        p = we * xr
        p = p[:, :128] + p[:, 128:]
        acc = p if acc is None else acc + p
    acc = acc[:, :64] + acc[:, 64:]
    msg_ref[...] = acc[:, :32] + acc[:, 32:]


def _messages(edge_attr, xj, w1, b1, w2, b2):
    e, d = edge_attr.shape
    h = xj.shape[1]
    blk = 4000
    grid = (e // blk,)
    rep = jnp.repeat(jnp.eye(h, dtype=jnp.bfloat16), h, axis=1)
    w2b = jnp.concatenate([w2, b2.reshape(1, h * h)],
                          axis=0).astype(jnp.bfloat16)
    return pl.pallas_call(
        functools.partial(_msg_body, h),
        grid=grid,
        in_specs=[
            pl.BlockSpec((blk, d), lambda i: (i, 0)),
            pl.BlockSpec((blk, h), lambda i: (i, 0)),
            pl.BlockSpec((d, h), lambda i: (0, 0)),
            pl.BlockSpec((1, h), lambda i: (0, 0)),
            pl.BlockSpec((h + 1, h * h), lambda i: (0, 0)),
            pl.BlockSpec((h, h * h), lambda i: (0, 0)),
        ],
        out_specs=pl.BlockSpec((blk, h), lambda i: (i, 0)),
        out_shape=jax.ShapeDtypeStruct((e, h), jnp.float32),
        compiler_params=pltpu.CompilerParams(
            dimension_semantics=("parallel",)),
    )(edge_attr, xj, w1.astype(jnp.bfloat16), b1.reshape(1, h), w2b, rep)


def _update_body(x_ref, p0_ref, p1_ref, root_ref, bias_ref, g_ref, b_ref,
                 out_ref):
    xb = x_ref[...]
    a = (p0_ref[...] + p1_ref[...]
         + jnp.dot(xb, root_ref[...], preferred_element_type=jnp.float32)
         + bias_ref[...])
    hh = _gelu(a) + xb
    mu = jnp.mean(hh, axis=1, keepdims=True)
    dlt = hh - mu
    var = jnp.mean(dlt * dlt, axis=1, keepdims=True)
    out_ref[...] = g_ref[...] * dlt * lax.rsqrt(var + 1e-5) + b_ref[...]


def _update(x, parts, root, bias, gamma, beta):
    n, h = x.shape
    blk = 2000
    grid = (n // blk,)
    off = n // blk
    row = lambda i: (i, 0)
    fix = lambda i: (0, 0)
    return pl.pallas_call(
        _update_body,
        grid=grid,
        in_specs=[
            pl.BlockSpec((blk, h), row),
            pl.BlockSpec((blk, h), row),
            pl.BlockSpec((blk, h), lambda i: (i + off, 0)),
            pl.BlockSpec((h, h), fix),
            pl.BlockSpec((1, h), fix),
            pl.BlockSpec((1, h), fix),
            pl.BlockSpec((1, h), fix),
        ],
        out_specs=pl.BlockSpec((blk, h), row),
        out_shape=jax.ShapeDtypeStruct((n, h), jnp.float32),
    )(x, parts, parts, root, bias.reshape(1, h), gamma.reshape(1, h),
      beta.reshape(1, h))


def kernel(x, edge_index, edge_attr, W1, b1, W2, b2, root, bias, gamma, beta):
    n = x.shape[0]
    e = edge_index.shape[1]
    ei2d = edge_index.reshape(2 * (e // _CH), _CH)
    xj = _gather(x, ei2d)
    msg = _messages(edge_attr, xj, W1, b1, W2, b2)
    parts = _scatter_add(msg, ei2d, n)
    return _update(x, parts, root, bias, gamma, beta)
